# baseline (speedup 1.0000x reference)
import jax
import jax.numpy as jnp
from jax.experimental import pallas as pl
from jax.experimental.pallas import tpu as pltpu


_BS = 8


def _inject_body(xv_ref, yv_ref, y_ref, x_ref, out_ref, a_ref, m_ref):
    H, n = y_ref.shape[1], xv_ref.shape[2]

    @pl.when(pl.program_id(0) == 0)
    def _build_onehots():
        riota = jax.lax.broadcasted_iota(jnp.int32, (H, n), 0)
        a_ref[...] = (riota == xv_ref[0]).astype(jnp.float32)
        m_ref[...] = (riota == yv_ref[0]).astype(jnp.float32)

    A = a_ref[...]
    M = m_ref[...]
    for b in range(y_ref.shape[0]):
        scaled = A * x_ref[b]
        D = jax.lax.dot_general(
            scaled, M, (((1,), (1,)), ((), ())),
            preferred_element_type=jnp.float32)
        out_ref[b] = y_ref[b] + D


def kernel(Y, X, x_idx, y_idx):
    Z, H, W = Y.shape
    n = X.shape[1]
    xv = x_idx.astype(jnp.int32).reshape(1, 1, n)
    yv = y_idx.astype(jnp.int32).reshape(1, 1, n)
    X3 = X.reshape(Z, 1, n)
    grid = (Z // _BS,)
    out = pl.pallas_call(
        _inject_body,
        grid=grid,
        in_specs=[
            pl.BlockSpec((1, 1, n), lambda z: (0, 0, 0)),
            pl.BlockSpec((1, 1, n), lambda z: (0, 0, 0)),
            pl.BlockSpec((_BS, H, W), lambda z: (z, 0, 0)),
            pl.BlockSpec((_BS, 1, n), lambda z: (z, 0, 0)),
        ],
        out_specs=pl.BlockSpec((_BS, H, W), lambda z: (z, 0, 0)),
        out_shape=jax.ShapeDtypeStruct((Z, H, W), jnp.float32),
        scratch_shapes=[
            pltpu.VMEM((H, n), jnp.float32),
            pltpu.VMEM((H, n), jnp.float32),
        ],
    )(xv, yv, Y, X3)
    return out
